# baseline (speedup 1.0000x reference)
import functools

import jax
import jax.numpy as jnp
from jax import lax
from jax.experimental import pallas as pl
from jax.experimental.pallas import tpu as pltpu
from jax.experimental.pallas import tpu_sc as plsc

_N = 49999
_ROWS = _N + 1
_BATCH = 4
_FEAT = 64
_LANES = 16


def _sc_gather(idx16, values_flat):
    mesh = plsc.VectorSubcoreMesh(core_axis_name="c", subcore_axis_name="s")

    @functools.partial(
        pl.kernel,
        out_type=jax.ShapeDtypeStruct((_LANES, _FEAT), jnp.float32),
        mesh=mesh,
        scratch_types=[
            pltpu.VMEM((_LANES,), jnp.int32),
            pltpu.VMEM((_LANES,), jnp.int32),
            pltpu.VMEM((_LANES, _FEAT), jnp.float32),
            pltpu.SemaphoreType.DMA,
        ],
        compiler_params=pltpu.CompilerParams(use_tc_tiling_on_sc=False),
    )
    def k(idx_hbm, vals_hbm, out_hbm, idx_v, rowid_v, rows_v, sem):
        wid = lax.axis_index("s") * 2 + lax.axis_index("c")

        @pl.when(wid == 0)
        def _():
            pltpu.sync_copy(idx_hbm, idx_v)
            lane = lax.iota(jnp.int32, _LANES)
            b = lane >> 2
            j = (lane >> 1) & 1
            half = lane & 1
            iv = idx_v[...]
            i0 = iv[0]
            i1 = iv[1]
            base = i0 + j * (i1 - i0)
            rowid_v[...] = b * (2 * _ROWS) + base + half * _ROWS
            pltpu.async_copy(vals_hbm.at[rowid_v], rows_v, sem).wait()
            pltpu.sync_copy(rows_v, out_hbm)

    return k(idx16, values_flat)


def kernel(indices, values):
    idx16 = jnp.zeros((_LANES,), jnp.int32).at[:2].set(indices.astype(jnp.int32))
    values_flat = values.reshape(_BATCH * 2 * _ROWS, _FEAT)
    out = _sc_gather(idx16, values_flat)
    return out.reshape(_BATCH, 2, 2 * _FEAT)


# speedup vs baseline: 2.5515x; 2.5515x over previous
import functools

import jax
import jax.numpy as jnp
from jax import lax
from jax.experimental import pallas as pl
from jax.experimental.pallas import tpu as pltpu
from jax.experimental.pallas import tpu_sc as plsc

_N = 49999
_ROWS = _N + 1
_BATCH = 4
_FEAT = 64
_LANES = 16


def _sc_gather(idx16, values_flat):
    mesh = plsc.VectorSubcoreMesh(core_axis_name="c", subcore_axis_name="s")

    @functools.partial(
        pl.kernel,
        out_type=jax.ShapeDtypeStruct((_LANES, _FEAT), jnp.float32),
        mesh=mesh,
        scratch_types=[
            pltpu.VMEM((_LANES,), jnp.int32),
            pltpu.VMEM((_LANES, _FEAT), jnp.float32),
            pltpu.SemaphoreType.DMA,
        ],
        compiler_params=pltpu.CompilerParams(use_tc_tiling_on_sc=True),
    )
    def k(idx_hbm, vals_hbm, out_hbm, idx_v, rows_v, sem):
        wid = lax.axis_index("s") * 2 + lax.axis_index("c")

        @pl.when(wid == 0)
        def _():
            pltpu.sync_copy(idx_hbm, idx_v)
            iv = idx_v[...]
            i0 = iv[0]
            i1 = iv[1]
            copies = []
            for lane in range(_LANES):
                b, j, half = lane >> 2, (lane >> 1) & 1, lane & 1
                row = (i1 if j else i0) + (b * 2 * _ROWS + half * _ROWS)
                copies.append(
                    pltpu.async_copy(
                        vals_hbm.at[pl.ds(row, 1)],
                        rows_v.at[pl.ds(lane, 1)],
                        sem,
                    )
                )
            for c in copies:
                c.wait()
            pltpu.sync_copy(rows_v, out_hbm)

    return k(idx16, values_flat)


def kernel(indices, values):
    idx16 = jnp.zeros((_LANES,), jnp.int32).at[:2].set(indices.astype(jnp.int32))
    values_flat = values.reshape(_BATCH * 2 * _ROWS, _FEAT)
    out = _sc_gather(idx16, values_flat)
    return out.reshape(_BATCH, 2, 2 * _FEAT)


# speedup vs baseline: 10.4332x; 4.0890x over previous
import functools

import jax
import jax.numpy as jnp
from jax import lax
from jax.experimental import pallas as pl
from jax.experimental.pallas import tpu as pltpu
from jax.experimental.pallas import tpu_sc as plsc

_N = 49999
_ROWS = _N + 1
_BATCH = 4
_FEAT = 64
_LANES = 16
_BLK = 128


def _sc_gather(idx16, values_t):
    mesh = plsc.VectorSubcoreMesh(core_axis_name="c", subcore_axis_name="s")

    @functools.partial(
        pl.kernel,
        out_type=jax.ShapeDtypeStruct((4 * _BATCH, _FEAT), jnp.float32),
        mesh=mesh,
        scratch_types=[
            pltpu.VMEM((_LANES,), jnp.int32),
            pltpu.VMEM((_BATCH, _FEAT, _BLK), jnp.float32),
            pltpu.VMEM((_BATCH, _FEAT), jnp.float32),
            pltpu.SemaphoreType.DMA,
        ],
        compiler_params=pltpu.CompilerParams(
            use_tc_tiling_on_sc=True,
            needs_layout_passes=False,
        ),
    )
    def k(idx_hbm, vals_hbm, out_hbm, idx_v, blk_v, col_v, sem):
        wid = lax.axis_index("s") * 2 + lax.axis_index("c")

        for q in range(4):

            @pl.when(wid == q)
            def _(q=q):
                pltpu.sync_copy(idx_hbm, idx_v)
                iv = idx_v[...]
                col = iv[q % 2] + (q // 2) * _ROWS
                base = (col // _BLK) * _BLK
                lane = col - base
                pltpu.async_copy(
                    vals_hbm.at[:, :, pl.ds(base, _BLK)], blk_v, sem
                ).wait()
                lane16 = jnp.broadcast_to(lane, (_LANES,))
                for b in range(_BATCH):
                    b16 = jnp.broadcast_to(jnp.int32(b), (_LANES,))
                    for fc in range(_FEAT // _LANES):
                        f16 = fc * _LANES + lax.iota(jnp.int32, _LANES)
                        col_v[b, pl.ds(fc * _LANES, _LANES)] = plsc.load_gather(
                            blk_v, [b16, f16, lane16]
                        )
                pltpu.sync_copy(col_v, out_hbm.at[pl.ds(q * _BATCH, _BATCH)])

    return k(idx16, values_t)


def kernel(indices, values):
    idx16 = jnp.zeros((_LANES,), jnp.int32).at[:2].set(indices.astype(jnp.int32))
    values_t = values.transpose(0, 2, 1)
    cols = _sc_gather(idx16, values_t)
    out = cols.reshape(2, 2, _BATCH, _FEAT)
    out = out.transpose(2, 1, 0, 3)
    return out.reshape(_BATCH, 2, 2 * _FEAT)


# speedup vs baseline: 10.6604x; 1.0218x over previous
import functools

import jax
import jax.numpy as jnp
from jax import lax
from jax.experimental import pallas as pl
from jax.experimental.pallas import tpu as pltpu
from jax.experimental.pallas import tpu_sc as plsc

_N = 49999
_ROWS = _N + 1
_BATCH = 4
_FEAT = 64
_LANES = 16
_BLK = 128


def _sc_gather(idx2, values_t):
    mesh = plsc.VectorSubcoreMesh(core_axis_name="c", subcore_axis_name="s")

    @functools.partial(
        pl.kernel,
        out_type=jax.ShapeDtypeStruct((_BATCH, 2, 2 * _FEAT), jnp.float32),
        mesh=mesh,
        scratch_types=[
            pltpu.VMEM((_LANES,), jnp.int32),
            pltpu.VMEM((2, _BATCH, _FEAT, _BLK), jnp.float32),
            pltpu.VMEM((_BATCH, 1, 2 * _FEAT), jnp.float32),
            pltpu.SemaphoreType.DMA,
        ],
        compiler_params=pltpu.CompilerParams(
            use_tc_tiling_on_sc=True,
            needs_layout_passes=False,
        ),
    )
    def k(idx_hbm, vals_hbm, out_hbm, idx_v, blk_v, row_v, sem):
        wid = lax.axis_index("s") * 2 + lax.axis_index("c")

        for j in range(2):

            @pl.when(wid == j)
            def _(j=j):
                pltpu.sync_copy(idx_hbm, idx_v.at[pl.ds(0, 2)])
                iv = idx_v[...]
                cj = iv[j]
                fetches = []
                for h in range(2):
                    col = cj + h * _ROWS
                    base = (col // _BLK) * _BLK
                    fetches.append(
                        (
                            pltpu.async_copy(
                                vals_hbm.at[:, :, pl.ds(base, _BLK)],
                                blk_v.at[h],
                                sem,
                            ),
                            col - base,
                        )
                    )
                for h, (fetch, lane) in enumerate(fetches):
                    fetch.wait()
                    lane16 = jnp.broadcast_to(lane, (_LANES,))
                    for b in range(_BATCH):
                        b16 = jnp.broadcast_to(jnp.int32(b), (_LANES,))
                        for fc in range(_FEAT // _LANES):
                            f16 = fc * _LANES + lax.iota(jnp.int32, _LANES)
                            row_v[b, 0, pl.ds(h * _FEAT + fc * _LANES, _LANES)] = (
                                plsc.load_gather(blk_v.at[h], [b16, f16, lane16])
                            )
                pltpu.sync_copy(row_v, out_hbm.at[:, pl.ds(j, 1), :])

    return k(idx2, values_t)


def kernel(indices, values):
    idx2 = indices.astype(jnp.int32)
    values_t = values.transpose(0, 2, 1)
    return _sc_gather(idx2, values_t)


# speedup vs baseline: 11.3272x; 1.0626x over previous
import functools

import jax
import jax.numpy as jnp
from jax import lax
from jax.experimental import pallas as pl
from jax.experimental.pallas import tpu as pltpu
from jax.experimental.pallas import tpu_sc as plsc

_N = 49999
_ROWS = _N + 1
_BATCH = 4
_FEAT = 64
_LANES = 16
_BLK = 128


def _sc_gather(idx2, values_t):
    mesh = plsc.VectorSubcoreMesh(
        core_axis_name="c", subcore_axis_name="s", num_cores=1, num_subcores=2
    )

    @functools.partial(
        pl.kernel,
        out_type=jax.ShapeDtypeStruct((_BATCH, 2, 2 * _FEAT), jnp.float32),
        mesh=mesh,
        scratch_types=[
            pltpu.VMEM((_LANES,), jnp.int32),
            pltpu.VMEM((2, _BATCH, _FEAT, _BLK), jnp.float32),
            pltpu.VMEM((_BATCH, 1, 2 * _FEAT), jnp.float32),
            pltpu.SemaphoreType.DMA,
        ],
        compiler_params=pltpu.CompilerParams(
            use_tc_tiling_on_sc=True,
            needs_layout_passes=False,
        ),
    )
    def k(idx_hbm, vals_hbm, out_hbm, idx_v, blk_v, row_v, sem):
        wid = lax.axis_index("s")

        for j in range(2):

            @pl.when(wid == j)
            def _(j=j):
                pltpu.sync_copy(idx_hbm, idx_v.at[pl.ds(0, 2)])
                iv = idx_v[...]
                cj = iv[j]
                fetches = []
                for h in range(2):
                    col = cj + h * _ROWS
                    base = (col // _BLK) * _BLK
                    fetches.append(
                        (
                            pltpu.async_copy(
                                vals_hbm.at[:, :, pl.ds(base, _BLK)],
                                blk_v.at[h],
                                sem,
                            ),
                            col - base,
                        )
                    )
                for h, (fetch, lane) in enumerate(fetches):
                    fetch.wait()
                    lane16 = jnp.broadcast_to(lane, (_LANES,))
                    for b in range(_BATCH):
                        b16 = jnp.broadcast_to(jnp.int32(b), (_LANES,))
                        for fc in range(_FEAT // _LANES):
                            f16 = fc * _LANES + lax.iota(jnp.int32, _LANES)
                            row_v[b, 0, pl.ds(h * _FEAT + fc * _LANES, _LANES)] = (
                                plsc.load_gather(blk_v.at[h], [b16, f16, lane16])
                            )
                pltpu.sync_copy(row_v, out_hbm.at[:, pl.ds(j, 1), :])

    return k(idx2, values_t)


def kernel(indices, values):
    idx2 = indices.astype(jnp.int32)
    values_t = values.transpose(0, 2, 1)
    return _sc_gather(idx2, values_t)
